# baseline (speedup 1.0000x reference)
import jax
import jax.numpy as jnp
from jax import lax
from jax.experimental import pallas as pl
from jax.experimental.pallas import tpu as pltpu
from jax.experimental.pallas import tpu_sc as plsc

NC = 2
NS = 16
LANES = 16
NW = NC * NS

B, L = 4096, 200
V, D = 1000000, 64
T = 2
EPS = 1e-12

N = B * L
RPW = N // NW
CHUNK = 256
NCH = RPW // CHUNK
SUB = 128
NSUB = CHUNK // SUB
GROUPS = CHUNK // LANES
DJ = D // LANES
NL = 8


def _emb_body(ids, tts, ctab, word, gamma, beta, out,
              idx_v, tvec_v, cbig, gb_v, xbufs, obufs, srow_v, sqrow_v,
              gsems, osems):
    wid = lax.axis_index("s") * NC + lax.axis_index("c")
    base = wid * RPW
    l0 = base // B

    pltpu.sync_copy(ids.at[pl.ds(base, RPW)], idx_v)
    pltpu.sync_copy(tts.at[pl.ds(base, RPW)], tvec_v)
    pltpu.sync_copy(ctab.at[0, pl.ds(l0, NL)], cbig.at[pl.ds(0, NL)])
    pltpu.sync_copy(ctab.at[1, pl.ds(l0, NL)], cbig.at[pl.ds(NL, NL)])
    pltpu.sync_copy(gamma, gb_v.at[0])
    pltpu.sync_copy(beta, gb_v.at[1])

    gvecs = [gb_v[0, pl.ds(j * LANES, LANES)] for j in range(DJ)]
    bvecs = [gb_v[1, pl.ds(j * LANES, LANES)] for j in range(DJ)]
    iota = lax.iota(jnp.int32, LANES)
    m15 = iota == (LANES - 1)
    kfulls = [jnp.full((LANES,), k, jnp.int32) for k in range(LANES)]
    ivecs = [(jnp.int32(16 * j) + iota) // 8 for j in range(DJ)]
    rvecs = [(jnp.int32(16 * j) + iota) % 8 for j in range(DJ)]

    def issue_gather(chunk, xb, sem):
        for j in range(NSUB):
            pltpu.async_copy(
                word.at[idx_v.at[pl.ds(chunk * CHUNK + j * SUB, SUB)]],
                xb.at[pl.ds(j * SUB, SUB)], sem)

    def drain_gather(xb, sem):
        pltpu.make_async_copy(word.at[pl.ds(0, CHUNK)], xb, sem).wait()

    def splat(vec, kfull):
        return vec.at[kfull].get(mode="promise_in_bounds")

    def compute(chunk, xb, ob):
        li = (base + chunk * CHUNK) // B - l0
        c0s = [cbig[li, pl.ds(j * LANES, LANES)] for j in range(DJ)]
        c1s = [cbig[NL + li, pl.ds(j * LANES, LANES)] for j in range(DJ)]

        @pl.loop(0, GROUPS)
        def _group(g):
            r0 = g * LANES
            tvec = tvec_v[pl.ds(chunk * CHUNK + r0, LANES)]
            for k in range(LANES):
                r = r0 + k
                tmask = splat(tvec, kfulls[k]) != 0
                xs = [xb[r, pl.ds(j * LANES, LANES)]
                      + jnp.where(tmask, c1s[j], c0s[j])
                      for j in range(DJ)]
                tot = (xs[0] + xs[1]) + (xs[2] + xs[3])
                sq = [x * x for x in xs]
                tot2 = (sq[0] + sq[1]) + (sq[2] + sq[3])
                plsc.store_scatter(srow_v, [kfulls[k]], plsc.cumsum(tot),
                                   mask=m15)
                plsc.store_scatter(sqrow_v, [kfulls[k]], plsc.cumsum(tot2),
                                   mask=m15)
                for j in range(DJ):
                    xb[r, pl.ds(j * LANES, LANES)] = xs[j]
            mean16 = srow_v[...] * (1.0 / D)
            ex216 = sqrow_v[...] * (1.0 / D)
            var = jnp.maximum(ex216 - mean16 * mean16, 0.0) + EPS
            yi = (jnp.int32(0x5F3759DF)
                  - (plsc.bitcast(var, jnp.int32) >> 1))
            y = plsc.bitcast(yi, jnp.float32)
            for _ in range(3):
                y = y * (1.5 - 0.5 * var * y * y)
            jjs = kfulls[1] * (g // 8)
            for k in range(LANES):
                r = r0 + k
                meank = splat(mean16, kfulls[k])
                invk = splat(y, kfulls[k])
                cs = kfulls[1] * ((g % 8) * LANES + k)
                for j in range(DJ):
                    xj = xb[r, pl.ds(j * LANES, LANES)]
                    yj = (xj - meank) * invk * gvecs[j] + bvecs[j]
                    plsc.store_scatter(ob, [ivecs[j], jjs, rvecs[j], cs],
                                       yj)

    def out_slice(chunk):
        gbase = base + chunk * CHUNK
        return out.at[gbase // B, :, pl.ds((gbase % B) // 128, CHUNK // 128)]

    def ob_src(ob):
        return ob.at[:, :, :, pl.ds(0, 128)]

    def drain_out(ob, sem):
        pltpu.make_async_copy(out_slice(0), ob_src(ob), sem).wait()

    issue_gather(0, xbufs[0], gsems[0])
    for b in range(2):
        pltpu.async_copy(ob_src(obufs[b]), out_slice(b), osems[b])

    @pl.loop(0, NCH, step=2)
    def _chunks(ci):
        for b in range(2):
            chunk = ci + b
            xb, sem, ob = xbufs[b], gsems[b], obufs[b]
            if b == 0:
                issue_gather(chunk + 1, xbufs[1], gsems[1])
            else:
                @pl.when(chunk + 1 < NCH)
                def _():
                    issue_gather(chunk + 1, xbufs[0], gsems[0])
            drain_gather(xb, sem)
            drain_out(ob, osems[b])
            compute(chunk, xb, ob)
            pltpu.async_copy(ob_src(ob), out_slice(chunk), osems[b])

    for b in range(2):
        drain_out(obufs[b], osems[b])


@jax.jit
def _emb(ids, tts, ctab, word, gamma, beta):
    mesh = plsc.VectorSubcoreMesh(core_axis_name="c", subcore_axis_name="s",
                                  num_cores=NC, num_subcores=NS)
    return pl.kernel(
        _emb_body,
        out_type=jax.ShapeDtypeStruct((L, D // 8, B // 128, 8, 128),
                                      jnp.float32),
        mesh=mesh,
        compiler_params=pltpu.CompilerParams(needs_layout_passes=False,
                                             use_tc_tiling_on_sc=False),
        scratch_types=[
            pltpu.VMEM((RPW,), jnp.int32),
            pltpu.VMEM((RPW,), jnp.int32),
            pltpu.VMEM((2 * NL, D), jnp.float32),
            pltpu.VMEM((2, D), jnp.float32),
            [pltpu.VMEM((CHUNK, D), jnp.float32)
             for _ in range(2)],
            [pltpu.VMEM((D // 8, CHUNK // 128, 8, 129),
                        jnp.float32)
             for _ in range(2)],
            pltpu.VMEM((LANES,), jnp.float32),
            pltpu.VMEM((LANES,), jnp.float32),
            [pltpu.SemaphoreType.DMA for _ in range(2)],
            [pltpu.SemaphoreType.DMA for _ in range(2)],
        ],
    )(ids, tts, ctab, word, gamma, beta)


def kernel(input_ids, token_type_ids, word_table, pos_table, type_table,
           gamma, beta):
    ids = input_ids.astype(jnp.int32).T.reshape(N)
    tts = token_type_ids.astype(jnp.int32).T.reshape(N)
    ctab = jnp.zeros((T, L + NL, D), jnp.float32)
    ctab = ctab.at[:, :L, :].set(type_table[:, None, :]
                                 + pos_table[None, :L, :])
    q = _emb(ids, tts, ctab, word_table, gamma, beta)
    return q.transpose(2, 4, 0, 1, 3).reshape(B, L, D)


# speedup vs baseline: 1.1849x; 1.1849x over previous
import jax
import jax.numpy as jnp
from jax import lax
from jax.experimental import pallas as pl
from jax.experimental.pallas import tpu as pltpu
from jax.experimental.pallas import tpu_sc as plsc

NC = 2
NS = 16
LANES = 16
NW = NC * NS

B, L = 4096, 200
V, D = 1000000, 64
T = 2
EPS = 1e-12

N = B * L
RPW = N // NW
CHUNK = 256
NCH = RPW // CHUNK
SUB = 128
NSUB = CHUNK // SUB
GROUPS = CHUNK // LANES
DJ = D // LANES
NL = 8


def _emb_body(ids, tts, ctab, word, gamma, beta, out,
              idx_v, tvec_v, cbig, gb_v, xbufs, srow_v, sqrow_v,
              gsems, osems):
    wid = lax.axis_index("s") * NC + lax.axis_index("c")
    base = wid * RPW
    l0 = base // B

    pltpu.sync_copy(ids.at[pl.ds(base, RPW)], idx_v)
    pltpu.sync_copy(tts.at[pl.ds(base, RPW)], tvec_v)
    pltpu.sync_copy(ctab.at[0, pl.ds(l0, NL)], cbig.at[pl.ds(0, NL)])
    pltpu.sync_copy(ctab.at[1, pl.ds(l0, NL)], cbig.at[pl.ds(NL, NL)])
    pltpu.sync_copy(gamma, gb_v.at[0])
    pltpu.sync_copy(beta, gb_v.at[1])

    gvecs = [gb_v[0, pl.ds(j * LANES, LANES)] for j in range(DJ)]
    bvecs = [gb_v[1, pl.ds(j * LANES, LANES)] for j in range(DJ)]
    iota = lax.iota(jnp.int32, LANES)
    m15 = iota == (LANES - 1)
    kfulls = [jnp.full((LANES,), k, jnp.int32) for k in range(LANES)]

    def issue_gather(chunk, xb, sem):
        for j in range(NSUB):
            pltpu.async_copy(
                word.at[idx_v.at[pl.ds(chunk * CHUNK + j * SUB, SUB)]],
                xb.at[pl.ds(j * SUB, SUB)], sem)

    def drain_gather(xb, sem):
        pltpu.make_async_copy(word.at[pl.ds(0, CHUNK)], xb, sem).wait()

    def splat(vec, kfull):
        return vec.at[kfull].get(mode="promise_in_bounds")

    def compute(chunk, xb):
        li = (base + chunk * CHUNK) // B - l0
        c0s = [cbig[li, pl.ds(j * LANES, LANES)] for j in range(DJ)]
        c1s = [cbig[NL + li, pl.ds(j * LANES, LANES)] for j in range(DJ)]

        @pl.loop(0, GROUPS)
        def _group(g):
            r0 = g * LANES
            tvec = tvec_v[pl.ds(chunk * CHUNK + r0, LANES)]
            for k in range(LANES):
                r = r0 + k
                tmask = splat(tvec, kfulls[k]) != 0
                xs = [xb[r, pl.ds(j * LANES, LANES)]
                      + jnp.where(tmask, c1s[j], c0s[j])
                      for j in range(DJ)]
                tot = (xs[0] + xs[1]) + (xs[2] + xs[3])
                sq = [x * x for x in xs]
                tot2 = (sq[0] + sq[1]) + (sq[2] + sq[3])
                plsc.store_scatter(srow_v, [kfulls[k]], plsc.cumsum(tot),
                                   mask=m15)
                plsc.store_scatter(sqrow_v, [kfulls[k]], plsc.cumsum(tot2),
                                   mask=m15)
                for j in range(DJ):
                    xb[r, pl.ds(j * LANES, LANES)] = xs[j]
            mean16 = srow_v[...] * (1.0 / D)
            ex216 = sqrow_v[...] * (1.0 / D)
            var = jnp.maximum(ex216 - mean16 * mean16, 0.0) + EPS
            yi = (jnp.int32(0x5F3759DF)
                  - (plsc.bitcast(var, jnp.int32) >> 1))
            y = plsc.bitcast(yi, jnp.float32)
            for _ in range(3):
                y = y * (1.5 - 0.5 * var * y * y)
            for k in range(LANES):
                r = r0 + k
                meank = splat(mean16, kfulls[k])
                invk = splat(y, kfulls[k])
                for j in range(DJ):
                    xj = xb[r, pl.ds(j * LANES, LANES)]
                    xb[r, pl.ds(j * LANES, LANES)] = (
                        (xj - meank) * invk * gvecs[j] + bvecs[j])

    def out_slice(chunk):
        gbase = base + chunk * CHUNK
        return out.at[gbase // B, pl.ds(gbase % B, CHUNK)]

    def drain_out(xb, sem):
        pltpu.make_async_copy(out_slice(0), xb, sem).wait()

    NB = 4
    issue_gather(0, xbufs[0], gsems[0])
    for b in range(1, NB):
        pltpu.async_copy(xbufs[b], out_slice(b), osems[b])

    @pl.loop(0, NCH, step=NB)
    def _chunks(ci):
        for b in range(NB):
            chunk = ci + b
            xb, sem = xbufs[b], gsems[b]
            nb = (b + 1) % NB

            def prefetch():
                drain_out(xbufs[nb], osems[nb])
                issue_gather(chunk + 1, xbufs[nb], gsems[nb])

            if b < NB - 1:
                prefetch()
            else:
                @pl.when(chunk + 1 < NCH)
                def _():
                    prefetch()
            drain_gather(xb, sem)
            compute(chunk, xb)
            pltpu.async_copy(xb, out_slice(chunk), osems[b])

    for b in range(NB):
        drain_out(xbufs[b], osems[b])


@jax.jit
def _emb(ids, tts, ctab, word, gamma, beta):
    mesh = plsc.VectorSubcoreMesh(core_axis_name="c", subcore_axis_name="s",
                                  num_cores=NC, num_subcores=NS)
    return pl.kernel(
        _emb_body,
        out_type=jax.ShapeDtypeStruct((L, B, D), jnp.float32),
        mesh=mesh,
        compiler_params=pltpu.CompilerParams(needs_layout_passes=False,
                                             use_tc_tiling_on_sc=False),
        scratch_types=[
            pltpu.VMEM((RPW,), jnp.int32),
            pltpu.VMEM((RPW,), jnp.int32),
            pltpu.VMEM((2 * NL, D), jnp.float32),
            pltpu.VMEM((2, D), jnp.float32),
            [pltpu.VMEM((CHUNK, D), jnp.float32)
             for _ in range(4)],
            pltpu.VMEM((LANES,), jnp.float32),
            pltpu.VMEM((LANES,), jnp.float32),
            [pltpu.SemaphoreType.DMA for _ in range(4)],
            [pltpu.SemaphoreType.DMA for _ in range(4)],
        ],
    )(ids, tts, ctab, word, gamma, beta)


def kernel(input_ids, token_type_ids, word_table, pos_table, type_table,
           gamma, beta):
    ids = input_ids.astype(jnp.int32).T.reshape(N)
    tts = token_type_ids.astype(jnp.int32).T.reshape(N)
    ctab = jnp.zeros((T, L + NL, D), jnp.float32)
    ctab = ctab.at[:, :L, :].set(type_table[:, None, :]
                                 + pos_table[None, :L, :])
    q = _emb(ids, tts, ctab, word_table, gamma, beta)
    return q.transpose(1, 0, 2)
